# baseline (speedup 1.0000x reference)
import functools

import jax
import jax.numpy as jnp
from jax import lax
from jax.experimental import pallas as pl
from jax.experimental.pallas import tpu as pltpu
from jax.experimental.pallas import tpu_sc as plsc

HIDDEN = 64
B, L = 4096, 200
N_TOKENS = B * L
N_CLASSES = 6

_info = plsc.get_sparse_core_info()
NC, NS = _info.num_cores, _info.num_subcores
NW = NC * NS

CHUNK = 512
IDXW = 128
NIDX = CHUNK // IDXW


def _table_body(emb_ref, w1_ref, b1_ref, w2_ref, b2_ref, out_ref):
    h = jnp.dot(emb_ref[...], w1_ref[...], preferred_element_type=jnp.float32)
    h = h + b1_ref[...]
    h = h * jax.nn.sigmoid(h)
    o = jnp.dot(h, w2_ref[...], preferred_element_type=jnp.float32)
    out_ref[...] = o + b2_ref[...]


def _mlp_table(emb, W1, b1, W2, b2):
    n = emb.shape[0]
    return pl.pallas_call(
        _table_body,
        out_shape=jax.ShapeDtypeStruct((n, HIDDEN), jnp.float32),
    )(emb, W1, b1.reshape(1, HIDDEN), W2, b2.reshape(1, HIDDEN))


def _make_gather():
    b_per_w = N_TOKENS // NW
    n_chunks = b_per_w // CHUNK
    n_outer = n_chunks // 2
    mesh = plsc.VectorSubcoreMesh(core_axis_name="c", subcore_axis_name="s")

    @functools.partial(
        pl.kernel,
        mesh=mesh,
        out_type=jax.ShapeDtypeStruct((N_TOKENS, HIDDEN), jnp.float32),
        scratch_types=[
            pltpu.VMEM((NIDX, IDXW), jnp.int32),
            pltpu.VMEM((NIDX, IDXW), jnp.int32),
            pltpu.VMEM((CHUNK, HIDDEN), jnp.float32),
            pltpu.VMEM((CHUNK, HIDDEN), jnp.float32),
            pltpu.SemaphoreType.DMA,
            pltpu.SemaphoreType.DMA,
            pltpu.SemaphoreType.DMA,
            pltpu.SemaphoreType.DMA,
            pltpu.SemaphoreType.DMA,
            pltpu.SemaphoreType.DMA,
        ],
        compiler_params=pltpu.CompilerParams(
            use_tc_tiling_on_sc=False, needs_layout_passes=False),
    )
    def gather_kernel(table_hbm, idx_hbm, out_hbm,
                      idx_a, idx_b, rows_a, rows_b,
                      si_a, si_b, sg_a, sg_b, so_a, so_b):
        wid = lax.axis_index("s") * NC + lax.axis_index("c")
        base = wid * b_per_w
        ibase = base // IDXW

        def fire_idx(k, buf, sem):
            pltpu.async_copy(
                idx_hbm.at[pl.ds(ibase + k * NIDX, NIDX)], buf, sem)

        def wait_idx(buf, sem):
            pltpu.make_async_copy(
                idx_hbm.at[pl.ds(ibase, NIDX)], buf, sem).wait()

        def fire_out(k, buf, sem):
            pltpu.async_copy(
                buf, out_hbm.at[pl.ds(base + k * CHUNK, CHUNK)], sem)

        def wait_out(buf, sem):
            pltpu.make_async_copy(
                buf, out_hbm.at[pl.ds(base, CHUNK)], sem).wait()

        fire_idx(0, idx_a, si_a)
        fire_idx(1, idx_b, si_b)

        def outer(kk, carry):
            for b, (idxv, rowsv, si, sg, so) in enumerate(
                    ((idx_a, rows_a, si_a, sg_a, so_a),
                     (idx_b, rows_b, si_b, sg_b, so_b))):
                k = kk * 2 + b
                wait_idx(idxv, si)

                @pl.when(kk > 0)
                def _drain():
                    wait_out(rowsv, so)

                for j in range(NIDX):
                    pltpu.async_copy(
                        table_hbm.at[idxv.at[j]],
                        rowsv.at[pl.ds(j * IDXW, IDXW)], sg)
                for j in range(NIDX):
                    pltpu.make_async_copy(
                        table_hbm.at[idxv.at[j]],
                        rowsv.at[pl.ds(j * IDXW, IDXW)], sg).wait()

                @pl.when(k + 2 < n_chunks)
                def _prefetch():
                    fire_idx(k + 2, idxv, si)

                fire_out(k, rowsv, so)
            return carry

        lax.fori_loop(0, n_outer, outer, 0)
        wait_out(rows_a, so_a)
        wait_out(rows_b, so_b)

    return gather_kernel


_gather = _make_gather()


def kernel(x, emb, W1, b1, W2, b2):
    table = _mlp_table(emb, W1, b1, W2, b2)
    idx = x.reshape(N_TOKENS // IDXW, IDXW).astype(jnp.int32)
    out = _gather(table, idx)
    return out.reshape(B, L, HIDDEN)


# speedup vs baseline: 6.8141x; 6.8141x over previous
import functools

import jax
import jax.numpy as jnp
from jax import lax
from jax.experimental import pallas as pl
from jax.experimental.pallas import tpu as pltpu
from jax.experimental.pallas import tpu_sc as plsc

HIDDEN = 64
B, L = 4096, 200
N_TOKENS = B * L
N_CLASSES = 6

_info = plsc.get_sparse_core_info()
NC, NS = _info.num_cores, _info.num_subcores
NW = NC * NS

CHUNK = 512
GROUPS = CHUNK // 16


def _table_body(emb_ref, w1_ref, b1_ref, w2_ref, b2_ref, out_ref):
    h = jnp.dot(emb_ref[...], w1_ref[...], preferred_element_type=jnp.float32)
    h = h + b1_ref[...]
    h = h * jax.nn.sigmoid(h)
    o = jnp.dot(h, w2_ref[...], preferred_element_type=jnp.float32)
    out_ref[...] = o + b2_ref[...]


def _mlp_table(emb, W1, b1, W2, b2):
    n = emb.shape[0]
    return pl.pallas_call(
        _table_body,
        out_shape=jax.ShapeDtypeStruct((n, HIDDEN), jnp.float32),
    )(emb, W1, b1.reshape(1, HIDDEN), W2, b2.reshape(1, HIDDEN))


def _make_gather():
    b_per_w = N_TOKENS // NW
    n_chunks = b_per_w // CHUNK
    n_outer = n_chunks // 2
    mesh = plsc.VectorSubcoreMesh(core_axis_name="c", subcore_axis_name="s")

    @functools.partial(
        pl.kernel,
        mesh=mesh,
        out_type=jax.ShapeDtypeStruct((N_TOKENS * HIDDEN,), jnp.float32),
        scratch_types=[
            pltpu.VMEM((N_CLASSES * HIDDEN,), jnp.float32),
            pltpu.VMEM((CHUNK,), jnp.int32),
            pltpu.VMEM((CHUNK,), jnp.int32),
            pltpu.VMEM((CHUNK * HIDDEN,), jnp.float32),
            pltpu.VMEM((CHUNK * HIDDEN,), jnp.float32),
            pltpu.SemaphoreType.DMA,
            pltpu.SemaphoreType.DMA,
            pltpu.SemaphoreType.DMA,
            pltpu.SemaphoreType.DMA,
        ],
        compiler_params=pltpu.CompilerParams(
            use_tc_tiling_on_sc=False, needs_layout_passes=False),
    )
    def gather_kernel(table_hbm, idx_hbm, out_hbm,
                      tbl_v, idx_a, idx_b, out_a, out_b,
                      si_a, si_b, so_a, so_b):
        wid = lax.axis_index("s") * NC + lax.axis_index("c")
        base = wid * b_per_w
        pltpu.sync_copy(table_hbm, tbl_v)

        def fire_idx(k, buf, sem):
            pltpu.async_copy(idx_hbm.at[pl.ds(base + k * CHUNK, CHUNK)], buf, sem)

        def wait_idx(buf, sem):
            pltpu.make_async_copy(
                idx_hbm.at[pl.ds(base, CHUNK)], buf, sem).wait()

        def fire_out(k, buf, sem):
            pltpu.async_copy(
                buf, out_hbm.at[pl.ds((base + k * CHUNK) * HIDDEN, CHUNK * HIDDEN)], sem)

        def wait_out(buf, sem):
            pltpu.make_async_copy(
                buf, out_hbm.at[pl.ds(base * HIDDEN, CHUNK * HIDDEN)], sem).wait()

        def compute(idx_ref, out_ref):
            def grp(g, carry):
                off16 = idx_ref[pl.ds(g * 16, 16)] * HIDDEN
                row0 = g * (16 * HIDDEN)
                for r in range(16):
                    src = off16[r]
                    dst = row0 + r * HIDDEN
                    for c in range(HIDDEN // 16):
                        out_ref[pl.ds(dst + c * 16, 16)] = (
                            tbl_v[pl.ds(src + c * 16, 16)])
                return carry
            lax.fori_loop(0, GROUPS, grp, 0)

        fire_idx(0, idx_a, si_a)
        fire_idx(1, idx_b, si_b)

        def outer(kk, carry):
            for b, (idxv, outv, si, so) in enumerate(
                    ((idx_a, out_a, si_a, so_a), (idx_b, out_b, si_b, so_b))):
                k = kk * 2 + b
                wait_idx(idxv, si)

                @pl.when(kk > 0)
                def _drain():
                    wait_out(outv, so)

                compute(idxv, outv)

                @pl.when(k + 2 < n_chunks)
                def _prefetch():
                    fire_idx(k + 2, idxv, si)

                fire_out(k, outv, so)
            return carry

        lax.fori_loop(0, n_outer, outer, 0)
        wait_out(out_a, so_a)
        wait_out(out_b, so_b)

    return gather_kernel


_gather = _make_gather()


def kernel(x, emb, W1, b1, W2, b2):
    table = _mlp_table(emb, W1, b1, W2, b2)
    idx = x.reshape(-1).astype(jnp.int32)
    out = _gather(table.reshape(-1), idx)
    return out.reshape(B, L, HIDDEN)
